# Initial kernel scaffold; baseline (speedup 1.0000x reference)
import jax
import jax.numpy as jnp
from jax.experimental import pallas as pl


def kernel(x, token_types, gamma, beta):
    raise NotImplementedError("write your pallas kernel here")


# speedup vs baseline: 3.0678x; 3.0678x over previous
import jax
import jax.numpy as jnp
from jax.experimental import pallas as pl

EPS = 1e-06
NUM_GROUPS = 4
BLOCK_T = 256


def _glnorm_kernel(x_ref, tt_ref, g_ref, b_ref, o_ref):
    x = x_ref[...]
    tt = tt_ref[...]
    d = x.shape[1]
    mean = jnp.mean(x, axis=1, keepdims=True)
    xc = x - mean
    var = jnp.mean(xc * xc, axis=1, keepdims=True)
    normed = xc * jax.lax.rsqrt(var + EPS)
    g = g_ref[...]
    b = b_ref[...]
    gg = g[NUM_GROUPS - 1][None, :]
    bb = b[NUM_GROUPS - 1][None, :]
    for k in range(NUM_GROUPS - 2, -1, -1):
        cond = tt == k
        gg = jnp.where(cond, g[k][None, :], gg)
        bb = jnp.where(cond, b[k][None, :], bb)
    o_ref[...] = normed * gg + bb


def kernel(x, token_types, gamma, beta):
    B, S, D = x.shape
    n_tok = B * S
    x2 = x.reshape(n_tok, D)
    tt2 = token_types.reshape(n_tok, 1).astype(jnp.int32)
    grid = (n_tok // BLOCK_T,)
    out = pl.pallas_call(
        _glnorm_kernel,
        grid=grid,
        in_specs=[
            pl.BlockSpec((BLOCK_T, D), lambda i: (i, 0)),
            pl.BlockSpec((BLOCK_T, 1), lambda i: (i, 0)),
            pl.BlockSpec((NUM_GROUPS, D), lambda i: (0, 0)),
            pl.BlockSpec((NUM_GROUPS, D), lambda i: (0, 0)),
        ],
        out_specs=pl.BlockSpec((BLOCK_T, D), lambda i: (i, 0)),
        out_shape=jax.ShapeDtypeStruct((n_tok, D), x.dtype),
    )(x2, tt2, gamma, beta)
    return out.reshape(B, S, D)


# speedup vs baseline: 3.7840x; 1.2334x over previous
import jax
import jax.numpy as jnp
from jax.experimental import pallas as pl

EPS = 1e-06
NUM_GROUPS = 4
BLOCK_T = 512


def _glnorm_kernel(x_ref, tt_ref, g_ref, b_ref, o_ref):
    x = x_ref[...]
    tt = tt_ref[...]
    d = x.shape[1]
    mean = jnp.mean(x, axis=1, keepdims=True)
    xc = x - mean
    var = jnp.mean(xc * xc, axis=1, keepdims=True)
    normed = xc * jax.lax.rsqrt(var + EPS)
    g = g_ref[...]
    b = b_ref[...]
    gg = g[NUM_GROUPS - 1][None, :]
    bb = b[NUM_GROUPS - 1][None, :]
    for k in range(NUM_GROUPS - 2, -1, -1):
        cond = tt == k
        gg = jnp.where(cond, g[k][None, :], gg)
        bb = jnp.where(cond, b[k][None, :], bb)
    o_ref[...] = normed * gg + bb


def kernel(x, token_types, gamma, beta):
    B, S, D = x.shape
    n_tok = B * S
    x2 = x.reshape(n_tok, D)
    tt2 = token_types.reshape(n_tok, 1).astype(jnp.int32)
    grid = (n_tok // BLOCK_T,)
    out = pl.pallas_call(
        _glnorm_kernel,
        grid=grid,
        in_specs=[
            pl.BlockSpec((BLOCK_T, D), lambda i: (i, 0)),
            pl.BlockSpec((BLOCK_T, 1), lambda i: (i, 0)),
            pl.BlockSpec((NUM_GROUPS, D), lambda i: (0, 0)),
            pl.BlockSpec((NUM_GROUPS, D), lambda i: (0, 0)),
        ],
        out_specs=pl.BlockSpec((BLOCK_T, D), lambda i: (i, 0)),
        out_shape=jax.ShapeDtypeStruct((n_tok, D), x.dtype),
    )(x2, tt2, gamma, beta)
    return out.reshape(B, S, D)


# speedup vs baseline: 4.2475x; 1.1225x over previous
import jax
import jax.numpy as jnp
from jax.experimental import pallas as pl

EPS = 1e-06
NUM_GROUPS = 4
BLOCK_T = 1024


def _glnorm_kernel(x_ref, tt_ref, g_ref, b_ref, o_ref):
    x = x_ref[...]
    tt = tt_ref[...]
    d = x.shape[1]
    mean = jnp.mean(x, axis=1, keepdims=True)
    xc = x - mean
    var = jnp.mean(xc * xc, axis=1, keepdims=True)
    normed = xc * jax.lax.rsqrt(var + EPS)
    g = g_ref[...]
    b = b_ref[...]
    gg = g[NUM_GROUPS - 1][None, :]
    bb = b[NUM_GROUPS - 1][None, :]
    for k in range(NUM_GROUPS - 2, -1, -1):
        cond = tt == k
        gg = jnp.where(cond, g[k][None, :], gg)
        bb = jnp.where(cond, b[k][None, :], bb)
    o_ref[...] = normed * gg + bb


def kernel(x, token_types, gamma, beta):
    B, S, D = x.shape
    n_tok = B * S
    x2 = x.reshape(n_tok, D)
    tt2 = token_types.reshape(n_tok, 1).astype(jnp.int32)
    grid = (n_tok // BLOCK_T,)
    out = pl.pallas_call(
        _glnorm_kernel,
        grid=grid,
        in_specs=[
            pl.BlockSpec((BLOCK_T, D), lambda i: (i, 0)),
            pl.BlockSpec((BLOCK_T, 1), lambda i: (i, 0)),
            pl.BlockSpec((NUM_GROUPS, D), lambda i: (0, 0)),
            pl.BlockSpec((NUM_GROUPS, D), lambda i: (0, 0)),
        ],
        out_specs=pl.BlockSpec((BLOCK_T, D), lambda i: (i, 0)),
        out_shape=jax.ShapeDtypeStruct((n_tok, D), x.dtype),
    )(x2, tt2, gamma, beta)
    return out.reshape(B, S, D)
